# baseline (speedup 1.0000x reference)
import functools
import math

import jax
import jax.numpy as jnp
from jax import lax
from jax.experimental import pallas as pl
from jax.experimental.pallas import tpu as pltpu
from jax.experimental.pallas import tpu_sc as plsc

_SIZE = 100000
_SMOOTHING = 0.1
_CONF = 1.0 - _SMOOTHING
_N = 1024
_FILL = _SMOOTHING / (_SIZE - 1)
_CONST = _N * ((_SIZE - 1) * _FILL * math.log(_FILL) + _CONF * math.log(_CONF))

_NC, _NS = 2, 16
_NW = _NC * _NS

_R_SC = 256
_R_TC = _N - _R_SC
_C_ALIGN = 99968
_RPW = _R_SC // _NW

_CH_SIZES = [3200] * 30 + [3968]
_CH_OFFS = [sum(_CH_SIZES[:k]) for k in range(len(_CH_SIZES))]
_CH_MAX = max(_CH_SIZES)


_TC_BR = 8
_TC_GRID = _R_TC // _TC_BR


def _tc_body(tgt3_ref, x_ref, sliv_ref, tgt2_ref, o_ref, acc1, acc2):
    i = pl.program_id(0)

    @pl.when(i == 0)
    def _init():
        x2 = sliv_ref[...]
        t2 = tgt2_ref[...]
        cols2 = jax.lax.broadcasted_iota(jnp.int32, x2.shape, 1) + _C_ALIGN
        rows2 = jax.lax.broadcasted_iota(jnp.int32, x2.shape, 0)
        sc_rows = rows2 >= _R_TC
        acc1[0] = jnp.sum(jnp.where(sc_rows, x2, jnp.float32(0.0)))
        acc2[0] = jnp.sum(jnp.where((cols2 == t2) & sc_rows, x2,
                                    jnp.float32(0.0)))

    x = x_ref[...]
    t = tgt3_ref[0, 0, :]
    cols = jax.lax.broadcasted_iota(jnp.int32, x.shape, 1)
    acc2[0] += jnp.sum(jnp.where(cols == t[:, None], x, jnp.float32(0.0)))
    acc1[0] += jnp.sum(x)

    @pl.when(i == _TC_GRID - 1)
    def _fin():
        o_ref[0, 0] = (jnp.float32(_CONST)
                       - jnp.float32(_FILL) * acc1[0]
                       - jnp.float32(_CONF - _FILL) * acc2[0])


def _tc_sum(x, sliver, tgt3, tgt2d):
    return pl.pallas_call(
        _tc_body,
        grid=(_TC_GRID,),
        in_specs=[
            pl.BlockSpec((1, 1, _TC_BR), lambda i: (i, 0, 0)),
            pl.BlockSpec((_TC_BR, _SIZE), lambda i: (i, 0)),
            pl.BlockSpec((_N, _SIZE - _C_ALIGN), lambda i: (0, 0)),
            pl.BlockSpec((_N, 1), lambda i: (0, 0)),
        ],
        out_specs=pl.BlockSpec(memory_space=pltpu.SMEM),
        out_shape=jax.ShapeDtypeStruct((1, 1), jnp.float32),
        scratch_shapes=[pltpu.SMEM((1,), jnp.float32),
                        pltpu.SMEM((1,), jnp.float32)],
        compiler_params=pltpu.CompilerParams(
            dimension_semantics=("arbitrary",),
        ),
    )(tgt3, x, sliver, tgt2d)


_sc_mesh = plsc.VectorSubcoreMesh(core_axis_name="c", subcore_axis_name="s")


@functools.partial(
    pl.kernel,
    mesh=_sc_mesh,
    out_type=jax.ShapeDtypeStruct((_NW, 16), jnp.float32),
    scratch_types=[
        pltpu.VMEM((16,), jnp.int32),
        pltpu.VMEM((2, 8, _CH_MAX), jnp.float32),
        pltpu.VMEM((16,), jnp.float32),
        pltpu.SemaphoreType.DMA,
        pltpu.SemaphoreType.DMA,
    ],
)
def _sc_part(x_hbm, tgt_hbm, out_hbm, tbuf, buf, stage, sem0, sem1):
    wid = lax.axis_index("s") * _NC + lax.axis_index("c")
    sems = (sem0, sem1)
    r0 = _R_TC + _RPW * wid

    pltpu.sync_copy(tgt_hbm.at[pl.ds(r0, _RPW)], tbuf.at[pl.ds(0, _RPW)])
    tvec = tbuf[pl.ds(0, 16)]
    ts = [tvec[r] for r in range(_RPW)]

    rows16 = lax.iota(jnp.int32, 16)
    zero16 = jnp.zeros((16,), jnp.float32)
    g = zero16
    accs = tuple(jnp.zeros((16,), jnp.float32) for _ in range(8))

    def _start(k):
        pltpu.async_copy(
            x_hbm.at[pl.ds(r0, 8), pl.ds(_CH_OFFS[k], _CH_SIZES[k])],
            buf.at[k % 2, :, pl.ds(0, _CH_SIZES[k])], sems[k % 2])

    _start(0)
    for k in range(len(_CH_SIZES)):
        if k + 1 < len(_CH_SIZES):
            _start(k + 1)
        slot = k % 2
        ch = _CH_SIZES[k]
        pltpu.make_async_copy(
            x_hbm.at[pl.ds(r0, 8), pl.ds(_CH_OFFS[k], ch)],
            buf.at[slot, :, pl.ds(0, ch)], sems[slot]).wait()

        def _ibody(jj, accs, slot=slot):
            j = pl.multiple_of(jj * 32, 32)
            out = []
            for r in range(8):
                out.append(accs[r] + buf[slot, r, pl.ds(j, 16)]
                           + buf[slot, r, pl.ds(j + 16, 16)])
            return tuple(out)

        accs = lax.fori_loop(0, ch // 32, _ibody, accs)

        for r in range(8):
            dt = ts[r] - jnp.int32(_CH_OFFS[k])
            l0 = jnp.minimum(jnp.maximum(dt & jnp.int32(-16), jnp.int32(0)),
                             jnp.int32(ch - 16))
            v = buf[slot, r, pl.ds(pl.multiple_of(l0, 16), 16)]
            lsel = jnp.where(dt >= 0,
                             jnp.where(dt < ch, dt & jnp.int32(15),
                                       jnp.int32(16)),
                             jnp.int32(16))
            g = g + jnp.where(rows16 == jnp.full((16,), lsel), v, zero16)

    acc = ((accs[0] + accs[1]) + (accs[2] + accs[3])
           + ((accs[4] + accs[5]) + (accs[6] + accs[7])))
    stage[...] = (jnp.float32(_FILL) * acc
                  + jnp.float32(_CONF - _FILL) * g)
    pltpu.sync_copy(stage, out_hbm.at[wid])


def kernel(x, target):
    tgt = target.astype(jnp.int32)
    sliver = lax.slice(x, (0, _C_ALIGN), (_N, _SIZE))
    tgt3 = lax.slice(tgt, (0,), (_R_TC,)).reshape(_TC_GRID, 1, _TC_BR)
    tc_out = _tc_sum(x, sliver, tgt3, tgt.reshape(_N, 1))
    sc_out = _sc_part(x, tgt)
    return (tc_out[0, 0] - jnp.sum(sc_out)).reshape(())
